# Initial kernel scaffold; baseline (speedup 1.0000x reference)
import jax
import jax.numpy as jnp
from jax.experimental import pallas as pl


def kernel(x, edge_index, W1, b1, W2, b2):
    raise NotImplementedError("write your pallas kernel here")


# speedup vs baseline: 32.2272x; 32.2272x over previous
import functools

import jax
import jax.numpy as jnp
from jax import lax
from jax.experimental import pallas as pl
from jax.experimental.pallas import tpu as pltpu
from jax.experimental.pallas import tpu_sc as plsc

N_NODES = 10000
N_EDGES = 320000
D_IN = 128
D_HID = 100
D_PAD = 112
N_CLASSES = 10

NC = 2
NS = 16
NW = NC * NS
E_PER_W = N_EDGES // NW
CHUNK = 80
NCHUNK = E_PER_W // CHUNK
NPAD1 = 10240
STRIPE1 = NPAD1 // NS
RSTRIPE = N_NODES // NS
RB = 125

_mesh = plsc.VectorSubcoreMesh(core_axis_name="c", subcore_axis_name="s")
_sc_params = pltpu.CompilerParams(needs_layout_passes=False,
                                  use_tc_tiling_on_sc=False)


@functools.partial(
    pl.kernel,
    out_type=jax.ShapeDtypeStruct((NC, NPAD1), jnp.float32),
    mesh=_mesh,
    compiler_params=_sc_params,
    scratch_types=[
        pltpu.VMEM((NCHUNK, CHUNK), jnp.int32),
        pltpu.VMEM((STRIPE1,), jnp.float32),
        pltpu.VMEM_SHARED((NPAD1,), jnp.float32),
    ],
)
def _deg_kernel(dst_hbm, out_hbm, dst_v, buf_v, deg_sh):
    cid = lax.axis_index("c")
    sid = lax.axis_index("s")
    wid = cid * NS + sid

    def zero_body(j, _):
        buf_v[pl.ds(j * 16, 16)] = jnp.zeros((16,), jnp.float32)
        return 0

    lax.fori_loop(0, STRIPE1 // 16, zero_body, 0)
    pltpu.sync_copy(buf_v, deg_sh.at[pl.ds(sid * STRIPE1, STRIPE1)])
    pltpu.sync_copy(dst_hbm.at[wid], dst_v)

    def ones_body(j, _):
        buf_v[pl.ds(j * 16, 16)] = jnp.full((16,), 1.0, jnp.float32)
        return 0

    lax.fori_loop(0, CHUNK // 16, ones_body, 0)
    plsc.subcore_barrier()

    def edge_body(ci, _):
        pltpu.sync_copy(buf_v.at[pl.ds(0, CHUNK)], deg_sh.at[dst_v.at[ci]],
                        add=True)
        return 0

    lax.fori_loop(0, NCHUNK, edge_body, 0)
    plsc.subcore_barrier()
    pltpu.sync_copy(deg_sh.at[pl.ds(sid * STRIPE1, STRIPE1)],
                    out_hbm.at[cid, pl.ds(sid * STRIPE1, STRIPE1)])


def _mm_body(degt_ref, x_ref, w1_ref, g_ref, dis_ref):
    deg = degt_ref[:, 0] + degt_ref[:, 1] + 1.0
    dis = lax.rsqrt(deg)
    h = jnp.dot(x_ref[...], w1_ref[...], preferred_element_type=jnp.float32)
    g_ref[...] = h * dis[:, None]
    dis_ref[...] = dis[:, None]


@functools.partial(
    pl.kernel,
    out_type=[
        jax.ShapeDtypeStruct((NC, N_NODES, D_PAD), jnp.float32),
        jax.ShapeDtypeStruct((NC, NPAD1), jnp.float32),
    ],
    mesh=_mesh,
    compiler_params=_sc_params,
    scratch_types=[
        pltpu.VMEM((NCHUNK, CHUNK), jnp.int32),
        pltpu.VMEM((NCHUNK, CHUNK), jnp.int32),
        pltpu.VMEM((N_NODES,), jnp.float32),
        pltpu.VMEM((RB, D_PAD), jnp.float32),
        pltpu.VMEM((CHUNK,), jnp.float32),
        pltpu.VMEM((STRIPE1,), jnp.float32),
        pltpu.VMEM_SHARED((N_NODES, D_PAD), jnp.float32),
        pltpu.VMEM_SHARED((NPAD1,), jnp.float32),
        pltpu.SemaphoreType.DMA,
    ],
)
def _edge_kernel(src_hbm, dst_hbm, g_hbm, dis_hbm, agg_out, s_out,
                 src_v, dst_v, dis_v, rows_v, vals_v, zbuf_v,
                 agg_sh, s_sh, sem):
    cid = lax.axis_index("c")
    sid = lax.axis_index("s")
    wid = cid * NS + sid

    def zrow_body(r, _):
        for k in range(D_PAD // 16):
            rows_v[r, pl.ds(k * 16, 16)] = jnp.zeros((16,), jnp.float32)
        return 0

    lax.fori_loop(0, RB, zrow_body, 0)

    def zs_body(j, _):
        zbuf_v[pl.ds(j * 16, 16)] = jnp.zeros((16,), jnp.float32)
        return 0

    lax.fori_loop(0, STRIPE1 // 16, zs_body, 0)

    for b in range(RSTRIPE // RB):
        pltpu.sync_copy(rows_v, agg_sh.at[pl.ds(sid * RSTRIPE + b * RB, RB)])
    pltpu.sync_copy(zbuf_v, s_sh.at[pl.ds(sid * STRIPE1, STRIPE1)])

    pltpu.sync_copy(src_hbm.at[wid], src_v)
    pltpu.sync_copy(dst_hbm.at[wid], dst_v)
    pltpu.sync_copy(dis_hbm, dis_v)
    plsc.subcore_barrier()

    def edge_body(ci, _):
        pltpu.async_copy(g_hbm.at[src_v.at[ci]], rows_v.at[pl.ds(0, CHUNK)],
                         sem).wait()
        for j in range(CHUNK // 16):
            d16 = dst_v[ci, pl.ds(j * 16, 16)]
            vals_v[pl.ds(j * 16, 16)] = plsc.load_gather(dis_v, [d16])
        pltpu.sync_copy(rows_v.at[pl.ds(0, CHUNK)], agg_sh.at[dst_v.at[ci]],
                        add=True)
        pltpu.sync_copy(vals_v, s_sh.at[src_v.at[ci]], add=True)
        return 0

    lax.fori_loop(0, NCHUNK, edge_body, 0)
    plsc.subcore_barrier()

    for b in range(RSTRIPE // RB):
        sl = pl.ds(sid * RSTRIPE + b * RB, RB)
        pltpu.sync_copy(agg_sh.at[sl], agg_out.at[cid, sl])
    pltpu.sync_copy(s_sh.at[pl.ds(sid * STRIPE1, STRIPE1)],
                    s_out.at[cid, pl.ds(sid * STRIPE1, STRIPE1)])


def _fin_body(aggp_ref, g_ref, dis_ref, st_ref, b1_ref, w2_ref, b2_ref,
              out_ref, acc_ref):
    i = pl.program_id(0)
    nb = pl.num_programs(0)
    agg = aggp_ref[0] + aggp_ref[1] + g_ref[...]
    dis = dis_ref[...]
    h1 = jnp.maximum(agg * dis + b1_ref[...], 0.0)
    srow = st_ref[:, 0] + st_ref[:, 1]
    c = dis[:, 0] * (srow + dis[:, 0])
    contrib = jnp.dot(c[None, :], h1, preferred_element_type=jnp.float32)

    @pl.when(i == 0)
    def _():
        acc_ref[...] = contrib

    @pl.when(i > 0)
    def _():
        acc_ref[...] = acc_ref[...] + contrib

    @pl.when(i == nb - 1)
    def _():
        v = acc_ref[...]
        pooled = (jnp.dot(v, w2_ref[...], preferred_element_type=jnp.float32)
                  / float(N_NODES)) + b2_ref[...]
        m = jnp.max(pooled, axis=1, keepdims=True)
        lse = jnp.log(jnp.sum(jnp.exp(pooled - m), axis=1, keepdims=True)) + m
        out_ref[...] = pooled - lse


def kernel(x, edge_index, W1, b1, W2, b2):
    f32 = jnp.float32
    src = edge_index[0].astype(jnp.int32).reshape(NW, NCHUNK, CHUNK)
    dst = edge_index[1].astype(jnp.int32).reshape(NW, NCHUNK, CHUNK)
    W1p = jnp.pad(W1.astype(f32), ((0, 0), (0, D_PAD - D_HID)))
    b1p = jnp.pad(b1.astype(f32), (0, D_PAD - D_HID)).reshape(1, D_PAD)
    W2p = jnp.pad(W2.astype(f32), ((0, D_PAD - D_HID), (0, 0)))
    b2r = b2.astype(f32).reshape(1, N_CLASSES)

    deg_parts = _deg_kernel(dst)
    deg_t = deg_parts[:, :N_NODES].T

    BR = 1000
    grid = (N_NODES // BR,)
    g, dis2 = pl.pallas_call(
        _mm_body,
        grid=grid,
        in_specs=[
            pl.BlockSpec((BR, NC), lambda i: (i, 0)),
            pl.BlockSpec((BR, D_IN), lambda i: (i, 0)),
            pl.BlockSpec((D_IN, D_PAD), lambda i: (0, 0)),
        ],
        out_specs=[
            pl.BlockSpec((BR, D_PAD), lambda i: (i, 0)),
            pl.BlockSpec((BR, 1), lambda i: (i, 0)),
        ],
        out_shape=[
            jax.ShapeDtypeStruct((N_NODES, D_PAD), f32),
            jax.ShapeDtypeStruct((N_NODES, 1), f32),
        ],
    )(deg_t, x.astype(f32), W1p)

    agg_parts, s_parts = _edge_kernel(src, dst, g, dis2.reshape(N_NODES))
    s_t = s_parts[:, :N_NODES].T

    out = pl.pallas_call(
        _fin_body,
        grid=grid,
        in_specs=[
            pl.BlockSpec((NC, BR, D_PAD), lambda i: (0, i, 0)),
            pl.BlockSpec((BR, D_PAD), lambda i: (i, 0)),
            pl.BlockSpec((BR, 1), lambda i: (i, 0)),
            pl.BlockSpec((BR, NC), lambda i: (i, 0)),
            pl.BlockSpec((1, D_PAD), lambda i: (0, 0)),
            pl.BlockSpec((D_PAD, N_CLASSES), lambda i: (0, 0)),
            pl.BlockSpec((1, N_CLASSES), lambda i: (0, 0)),
        ],
        out_specs=pl.BlockSpec((1, N_CLASSES), lambda i: (0, 0)),
        out_shape=jax.ShapeDtypeStruct((1, N_CLASSES), f32),
        scratch_shapes=[pltpu.VMEM((1, D_PAD), f32)],
    )(agg_parts, g, dis2, s_t, b1p, W2p, b2r)
    return out


# speedup vs baseline: 45.6707x; 1.4171x over previous
import functools

import jax
import jax.numpy as jnp
from jax import lax
from jax.experimental import pallas as pl
from jax.experimental.pallas import tpu as pltpu
from jax.experimental.pallas import tpu_sc as plsc

N_NODES = 10000
N_EDGES = 320000
D_IN = 128
D_HID = 100
D_PAD = 112
N_CLASSES = 10

NC = 2
NS = 16
NW = NC * NS
E_PER_W = N_EDGES // NW
CHUNK = 80
NCHUNK = E_PER_W // CHUNK
NPAD1 = 10240
STRIPE1 = NPAD1 // NS
RSTRIPE = N_NODES // NS
RB = 125

_mesh = plsc.VectorSubcoreMesh(core_axis_name="c", subcore_axis_name="s")
_sc_params = pltpu.CompilerParams(needs_layout_passes=False,
                                  use_tc_tiling_on_sc=False)


@functools.partial(
    pl.kernel,
    out_type=jax.ShapeDtypeStruct((NC, NPAD1), jnp.float32),
    mesh=_mesh,
    compiler_params=_sc_params,
    scratch_types=[
        pltpu.VMEM((NCHUNK, CHUNK), jnp.int32),
        pltpu.VMEM((STRIPE1,), jnp.float32),
        pltpu.VMEM_SHARED((NPAD1,), jnp.float32),
    ],
)
def _deg_kernel(dst_hbm, out_hbm, dst_v, buf_v, deg_sh):
    cid = lax.axis_index("c")
    sid = lax.axis_index("s")
    wid = cid * NS + sid

    def zero_body(j, _):
        buf_v[pl.ds(j * 16, 16)] = jnp.zeros((16,), jnp.float32)
        return 0

    lax.fori_loop(0, STRIPE1 // 16, zero_body, 0)
    pltpu.sync_copy(buf_v, deg_sh.at[pl.ds(sid * STRIPE1, STRIPE1)])
    pltpu.sync_copy(dst_hbm.at[wid], dst_v)

    def ones_body(j, _):
        buf_v[pl.ds(j * 16, 16)] = jnp.full((16,), 1.0, jnp.float32)
        return 0

    lax.fori_loop(0, CHUNK // 16, ones_body, 0)
    plsc.subcore_barrier()

    def edge_body(ci, _):
        pltpu.sync_copy(buf_v.at[pl.ds(0, CHUNK)], deg_sh.at[dst_v.at[ci]],
                        add=True)
        return 0

    lax.fori_loop(0, NCHUNK, edge_body, 0)
    plsc.subcore_barrier()
    pltpu.sync_copy(deg_sh.at[pl.ds(sid * STRIPE1, STRIPE1)],
                    out_hbm.at[cid, pl.ds(sid * STRIPE1, STRIPE1)])


def _mm_body(degt_ref, x_ref, w1_ref, g_ref, dis_ref):
    deg = degt_ref[:, 0] + degt_ref[:, 1] + 1.0
    dis = lax.rsqrt(deg)
    h = jnp.dot(x_ref[...], w1_ref[...], preferred_element_type=jnp.float32)
    g_ref[...] = h * dis[:, None]
    dis_ref[...] = dis[:, None]


@functools.partial(
    pl.kernel,
    out_type=[
        jax.ShapeDtypeStruct((NC, N_NODES, D_PAD), jnp.float32),
        jax.ShapeDtypeStruct((NC, NPAD1), jnp.float32),
    ],
    mesh=_mesh,
    compiler_params=_sc_params,
    scratch_types=[
        pltpu.VMEM((NCHUNK, CHUNK), jnp.int32),
        pltpu.VMEM((NCHUNK, CHUNK), jnp.int32),
        pltpu.VMEM((N_NODES,), jnp.float32),
        pltpu.VMEM((CHUNK, D_PAD), jnp.float32),
        pltpu.VMEM((CHUNK, D_PAD), jnp.float32),
        pltpu.VMEM((CHUNK,), jnp.float32),
        pltpu.VMEM((STRIPE1,), jnp.float32),
        pltpu.VMEM_SHARED((N_NODES, D_PAD), jnp.float32),
        pltpu.VMEM_SHARED((NPAD1,), jnp.float32),
        pltpu.SemaphoreType.DMA,
        pltpu.SemaphoreType.DMA,
    ],
)
def _edge_kernel(src_hbm, dst_hbm, g_hbm, dis_hbm, agg_out, s_out,
                 src_v, dst_v, dis_v, rows_a, rows_b, vals_v, zbuf_v,
                 agg_sh, s_sh, sem_a, sem_b):
    cid = lax.axis_index("c")
    sid = lax.axis_index("s")
    wid = cid * NS + sid

    def zrow_body(r, _):
        for k in range(D_PAD // 16):
            rows_a[r, pl.ds(k * 16, 16)] = jnp.zeros((16,), jnp.float32)
            rows_b[r, pl.ds(k * 16, 16)] = jnp.zeros((16,), jnp.float32)
        return 0

    lax.fori_loop(0, CHUNK, zrow_body, 0)

    def zs_body(j, _):
        zbuf_v[pl.ds(j * 16, 16)] = jnp.zeros((16,), jnp.float32)
        return 0

    lax.fori_loop(0, STRIPE1 // 16, zs_body, 0)

    for b in range(RSTRIPE // CHUNK):
        pltpu.sync_copy(rows_a,
                        agg_sh.at[pl.ds(sid * RSTRIPE + b * CHUNK, CHUNK)])
    pltpu.sync_copy(
        rows_b.at[pl.ds(0, RSTRIPE - (RSTRIPE // CHUNK) * CHUNK)],
        agg_sh.at[pl.ds(sid * RSTRIPE + (RSTRIPE // CHUNK) * CHUNK,
                        RSTRIPE - (RSTRIPE // CHUNK) * CHUNK)])
    pltpu.sync_copy(zbuf_v, s_sh.at[pl.ds(sid * STRIPE1, STRIPE1)])

    pltpu.sync_copy(src_hbm.at[wid], src_v)
    pltpu.sync_copy(dst_hbm.at[wid], dst_v)
    pltpu.sync_copy(dis_hbm, dis_v)
    plsc.subcore_barrier()

    def _drain(ci, buf, sem):
        pltpu.make_async_copy(g_hbm.at[src_v.at[ci]], buf, sem).wait()

    def _consume(ci, buf):
        for j in range(CHUNK // 16):
            d16 = dst_v[ci, pl.ds(j * 16, 16)]
            vals_v[pl.ds(j * 16, 16)] = plsc.load_gather(dis_v, [d16])
        pltpu.sync_copy(buf, agg_sh.at[dst_v.at[ci]], add=True)
        pltpu.sync_copy(vals_v, s_sh.at[src_v.at[ci]], add=True)

    pltpu.async_copy(g_hbm.at[src_v.at[0]], rows_a, sem_a)

    def edge_body(k, _):
        c0 = 2 * k
        pltpu.async_copy(g_hbm.at[src_v.at[c0 + 1]], rows_b, sem_b)
        _drain(c0, rows_a, sem_a)
        _consume(c0, rows_a)
        pltpu.async_copy(g_hbm.at[src_v.at[c0 + 2]], rows_a, sem_a)
        _drain(c0 + 1, rows_b, sem_b)
        _consume(c0 + 1, rows_b)
        return 0

    lax.fori_loop(0, (NCHUNK - 1) // 2, edge_body, 0)
    _drain(NCHUNK - 1, rows_a, sem_a)
    _consume(NCHUNK - 1, rows_a)
    plsc.subcore_barrier()

    sl = pl.ds(sid * RSTRIPE, RSTRIPE)
    pltpu.sync_copy(agg_sh.at[sl], agg_out.at[cid, sl])
    pltpu.sync_copy(s_sh.at[pl.ds(sid * STRIPE1, STRIPE1)],
                    s_out.at[cid, pl.ds(sid * STRIPE1, STRIPE1)])


def _fin_body(aggp_ref, g_ref, dis_ref, st_ref, b1_ref, w2_ref, b2_ref,
              out_ref, acc_ref):
    i = pl.program_id(0)
    nb = pl.num_programs(0)
    agg = aggp_ref[0] + aggp_ref[1] + g_ref[...]
    dis = dis_ref[...]
    h1 = jnp.maximum(agg * dis + b1_ref[...], 0.0)
    srow = st_ref[:, 0] + st_ref[:, 1]
    c = dis[:, 0] * (srow + dis[:, 0])
    contrib = jnp.dot(c[None, :], h1, preferred_element_type=jnp.float32)

    @pl.when(i == 0)
    def _():
        acc_ref[...] = contrib

    @pl.when(i > 0)
    def _():
        acc_ref[...] = acc_ref[...] + contrib

    @pl.when(i == nb - 1)
    def _():
        v = acc_ref[...]
        pooled = (jnp.dot(v, w2_ref[...], preferred_element_type=jnp.float32)
                  / float(N_NODES)) + b2_ref[...]
        m = jnp.max(pooled, axis=1, keepdims=True)
        lse = jnp.log(jnp.sum(jnp.exp(pooled - m), axis=1, keepdims=True)) + m
        out_ref[...] = pooled - lse


def kernel(x, edge_index, W1, b1, W2, b2):
    f32 = jnp.float32
    src = edge_index[0].astype(jnp.int32).reshape(NW, NCHUNK, CHUNK)
    dst = edge_index[1].astype(jnp.int32).reshape(NW, NCHUNK, CHUNK)
    W1p = jnp.pad(W1.astype(f32), ((0, 0), (0, D_PAD - D_HID)))
    b1p = jnp.pad(b1.astype(f32), (0, D_PAD - D_HID)).reshape(1, D_PAD)
    W2p = jnp.pad(W2.astype(f32), ((0, D_PAD - D_HID), (0, 0)))
    b2r = b2.astype(f32).reshape(1, N_CLASSES)

    deg_parts = _deg_kernel(dst)
    deg_t = deg_parts[:, :N_NODES].T

    BR = 1000
    grid = (N_NODES // BR,)
    g, dis2 = pl.pallas_call(
        _mm_body,
        grid=grid,
        in_specs=[
            pl.BlockSpec((BR, NC), lambda i: (i, 0)),
            pl.BlockSpec((BR, D_IN), lambda i: (i, 0)),
            pl.BlockSpec((D_IN, D_PAD), lambda i: (0, 0)),
        ],
        out_specs=[
            pl.BlockSpec((BR, D_PAD), lambda i: (i, 0)),
            pl.BlockSpec((BR, 1), lambda i: (i, 0)),
        ],
        out_shape=[
            jax.ShapeDtypeStruct((N_NODES, D_PAD), f32),
            jax.ShapeDtypeStruct((N_NODES, 1), f32),
        ],
    )(deg_t, x.astype(f32), W1p)

    agg_parts, s_parts = _edge_kernel(src, dst, g, dis2.reshape(N_NODES))
    s_t = s_parts[:, :N_NODES].T

    out = pl.pallas_call(
        _fin_body,
        grid=grid,
        in_specs=[
            pl.BlockSpec((NC, BR, D_PAD), lambda i: (0, i, 0)),
            pl.BlockSpec((BR, D_PAD), lambda i: (i, 0)),
            pl.BlockSpec((BR, 1), lambda i: (i, 0)),
            pl.BlockSpec((BR, NC), lambda i: (i, 0)),
            pl.BlockSpec((1, D_PAD), lambda i: (0, 0)),
            pl.BlockSpec((D_PAD, N_CLASSES), lambda i: (0, 0)),
            pl.BlockSpec((1, N_CLASSES), lambda i: (0, 0)),
        ],
        out_specs=pl.BlockSpec((1, N_CLASSES), lambda i: (0, 0)),
        out_shape=jax.ShapeDtypeStruct((1, N_CLASSES), f32),
        scratch_shapes=[pltpu.VMEM((1, D_PAD), f32)],
    )(agg_parts, g, dis2, s_t, b1p, W2p, b2r)
    return out


# speedup vs baseline: 50.8378x; 1.1131x over previous
import functools

import jax
import jax.numpy as jnp
from jax import lax
from jax.experimental import pallas as pl
from jax.experimental.pallas import tpu as pltpu
from jax.experimental.pallas import tpu_sc as plsc

N_NODES = 10000
N_EDGES = 320000
D_IN = 128
D_HID = 100
D_PAD = 112
N_CLASSES = 10

NC = 2
NS = 16
NW = NC * NS
E_PER_W = N_EDGES // NW
CHUNK = 80
NCHUNK = E_PER_W // CHUNK
NPAD1 = 10240
STRIPE1 = NPAD1 // NS
RSTRIPE = N_NODES // NS
NBUF = 3
LOOKAHEAD = 2

_mesh = plsc.VectorSubcoreMesh(core_axis_name="c", subcore_axis_name="s")
_sc_params = pltpu.CompilerParams(needs_layout_passes=False,
                                  use_tc_tiling_on_sc=False)


@functools.partial(
    pl.kernel,
    out_type=jax.ShapeDtypeStruct((NC, NPAD1), jnp.float32),
    mesh=_mesh,
    compiler_params=_sc_params,
    scratch_types=[
        pltpu.VMEM((NCHUNK, CHUNK), jnp.int32),
        pltpu.VMEM((STRIPE1,), jnp.float32),
        pltpu.VMEM_SHARED((NPAD1,), jnp.float32),
    ],
)
def _deg_kernel(dst_hbm, out_hbm, dst_v, buf_v, deg_sh):
    cid = lax.axis_index("c")
    sid = lax.axis_index("s")
    wid = cid * NS + sid

    def zero_body(j, _):
        buf_v[pl.ds(j * 16, 16)] = jnp.zeros((16,), jnp.float32)
        return 0

    lax.fori_loop(0, STRIPE1 // 16, zero_body, 0)
    pltpu.sync_copy(buf_v, deg_sh.at[pl.ds(sid * STRIPE1, STRIPE1)])
    pltpu.sync_copy(dst_hbm.at[wid], dst_v)

    def ones_body(j, _):
        buf_v[pl.ds(j * 16, 16)] = jnp.full((16,), 1.0, jnp.float32)
        return 0

    lax.fori_loop(0, CHUNK // 16, ones_body, 0)
    plsc.subcore_barrier()

    def edge_body(ci, _):
        pltpu.sync_copy(buf_v.at[pl.ds(0, CHUNK)], deg_sh.at[dst_v.at[ci]],
                        add=True)
        return 0

    lax.fori_loop(0, NCHUNK, edge_body, 0)
    plsc.subcore_barrier()
    pltpu.sync_copy(deg_sh.at[pl.ds(sid * STRIPE1, STRIPE1)],
                    out_hbm.at[cid, pl.ds(sid * STRIPE1, STRIPE1)])


def _mm_body(x_ref, w1_ref, h_ref):
    h_ref[...] = jnp.dot(x_ref[...], w1_ref[...],
                         preferred_element_type=jnp.float32)


def _scale_body(degt_ref, h_ref, g_ref, dis_ref):
    deg = degt_ref[:, 0] + degt_ref[:, 1] + 1.0
    dis = lax.rsqrt(deg)
    g_ref[...] = h_ref[...] * dis[:, None]
    dis_ref[...] = dis[:, None]


@functools.partial(
    pl.kernel,
    out_type=[
        jax.ShapeDtypeStruct((NC, N_NODES, D_PAD), jnp.float32),
        jax.ShapeDtypeStruct((NC, NPAD1), jnp.float32),
    ],
    mesh=_mesh,
    compiler_params=_sc_params,
    scratch_types=[
        pltpu.VMEM((NCHUNK, CHUNK), jnp.int32),
        pltpu.VMEM((NCHUNK, CHUNK), jnp.int32),
        pltpu.VMEM((N_NODES,), jnp.float32),
        [pltpu.VMEM((CHUNK, D_PAD), jnp.float32) for _ in range(NBUF)],
        [pltpu.VMEM((CHUNK,), jnp.float32) for _ in range(NBUF)],
        pltpu.VMEM((STRIPE1,), jnp.float32),
        pltpu.VMEM_SHARED((N_NODES, D_PAD), jnp.float32),
        pltpu.VMEM_SHARED((NPAD1,), jnp.float32),
        [pltpu.SemaphoreType.DMA for _ in range(NBUF)],
        [pltpu.SemaphoreType.DMA for _ in range(NBUF)],
    ],
)
def _edge_kernel(src_hbm, dst_hbm, g_hbm, dis_hbm, agg_out, s_out,
                 src_v, dst_v, dis_v, bufs, vals, zbuf_v,
                 agg_sh, s_sh, gsem, ssem):
    cid = lax.axis_index("c")
    sid = lax.axis_index("s")
    wid = cid * NS + sid

    def zrow_body(r, _):
        for k in range(D_PAD // 16):
            bufs[0][r, pl.ds(k * 16, 16)] = jnp.zeros((16,), jnp.float32)
        return 0

    lax.fori_loop(0, CHUNK, zrow_body, 0)

    def zs_body(j, _):
        zbuf_v[pl.ds(j * 16, 16)] = jnp.zeros((16,), jnp.float32)
        return 0

    lax.fori_loop(0, STRIPE1 // 16, zs_body, 0)

    for b in range(RSTRIPE // CHUNK):
        pltpu.sync_copy(bufs[0],
                        agg_sh.at[pl.ds(sid * RSTRIPE + b * CHUNK, CHUNK)])
    _tail = RSTRIPE - (RSTRIPE // CHUNK) * CHUNK
    pltpu.sync_copy(
        bufs[0].at[pl.ds(0, _tail)],
        agg_sh.at[pl.ds(sid * RSTRIPE + (RSTRIPE // CHUNK) * CHUNK, _tail)])
    pltpu.sync_copy(zbuf_v, s_sh.at[pl.ds(sid * STRIPE1, STRIPE1)])

    pltpu.sync_copy(src_hbm.at[wid], src_v)
    pltpu.sync_copy(dst_hbm.at[wid], dst_v)
    pltpu.sync_copy(dis_hbm, dis_v)
    plsc.subcore_barrier()

    def _wait_pair(b):
        pltpu.make_async_copy(bufs[b], agg_sh.at[dst_v.at[0]], ssem[b]).wait()
        pltpu.make_async_copy(vals[b], s_sh.at[src_v.at[0]], ssem[b]).wait()

    def _consume(c, b):
        pltpu.make_async_copy(g_hbm.at[src_v.at[c]], bufs[b], gsem[b]).wait()
        for j in range(CHUNK // 16):
            d16 = dst_v[c, pl.ds(j * 16, 16)]
            vals[b][pl.ds(j * 16, 16)] = plsc.load_gather(dis_v, [d16])
        pltpu.async_copy(bufs[b], agg_sh.at[dst_v.at[c]], ssem[b], add=True)
        pltpu.async_copy(vals[b], s_sh.at[src_v.at[c]], ssem[b], add=True)

    for c in range(LOOKAHEAD):
        pltpu.async_copy(g_hbm.at[src_v.at[c]], bufs[c % NBUF], gsem[c % NBUF])

    def edge_body(k, _):
        for b in range(NBUF):
            c = k * NBUF + b
            _consume(c, b)
            cn = c + LOOKAHEAD
            bn = (b + LOOKAHEAD) % NBUF

            @pl.when(cn < NCHUNK)
            def _():
                @pl.when(cn >= NBUF)
                def _():
                    _wait_pair(bn)

                pltpu.async_copy(g_hbm.at[src_v.at[cn]], bufs[bn], gsem[bn])

        return 0

    nfull = NCHUNK // NBUF
    lax.fori_loop(0, nfull, edge_body, 0)
    for b in range(NCHUNK - nfull * NBUF):
        _consume(nfull * NBUF + b, b)
    for b in range(NBUF):
        _wait_pair(b)
    plsc.subcore_barrier()

    sl = pl.ds(sid * RSTRIPE, RSTRIPE)
    pltpu.sync_copy(agg_sh.at[sl], agg_out.at[cid, sl])
    pltpu.sync_copy(s_sh.at[pl.ds(sid * STRIPE1, STRIPE1)],
                    s_out.at[cid, pl.ds(sid * STRIPE1, STRIPE1)])


def _fin_body(aggp_ref, g_ref, dis_ref, st_ref, b1_ref, w2_ref, b2_ref,
              out_ref, acc_ref):
    i = pl.program_id(0)
    nb = pl.num_programs(0)
    agg = aggp_ref[0] + aggp_ref[1] + g_ref[...]
    dis = dis_ref[...]
    h1 = jnp.maximum(agg * dis + b1_ref[...], 0.0)
    srow = st_ref[:, 0] + st_ref[:, 1]
    c = dis[:, 0] * (srow + dis[:, 0])
    contrib = jnp.dot(c[None, :], h1, preferred_element_type=jnp.float32)

    @pl.when(i == 0)
    def _():
        acc_ref[...] = contrib

    @pl.when(i > 0)
    def _():
        acc_ref[...] = acc_ref[...] + contrib

    @pl.when(i == nb - 1)
    def _():
        v = acc_ref[...]
        pooled = (jnp.dot(v, w2_ref[...], preferred_element_type=jnp.float32)
                  / float(N_NODES)) + b2_ref[...]
        m = jnp.max(pooled, axis=1, keepdims=True)
        lse = jnp.log(jnp.sum(jnp.exp(pooled - m), axis=1, keepdims=True)) + m
        out_ref[...] = pooled - lse


def kernel(x, edge_index, W1, b1, W2, b2):
    f32 = jnp.float32
    src = edge_index[0].astype(jnp.int32).reshape(NW, NCHUNK, CHUNK)
    dst = edge_index[1].astype(jnp.int32).reshape(NW, NCHUNK, CHUNK)
    W1p = jnp.pad(W1.astype(f32), ((0, 0), (0, D_PAD - D_HID)))
    b1p = jnp.pad(b1.astype(f32), (0, D_PAD - D_HID)).reshape(1, D_PAD)
    W2p = jnp.pad(W2.astype(f32), ((0, D_PAD - D_HID), (0, 0)))
    b2r = b2.astype(f32).reshape(1, N_CLASSES)

    deg_parts = _deg_kernel(dst)
    deg_t = deg_parts[:, :N_NODES].T

    BR = 1000
    grid = (N_NODES // BR,)
    h = pl.pallas_call(
        _mm_body,
        grid=grid,
        in_specs=[
            pl.BlockSpec((BR, D_IN), lambda i: (i, 0)),
            pl.BlockSpec((D_IN, D_PAD), lambda i: (0, 0)),
        ],
        out_specs=pl.BlockSpec((BR, D_PAD), lambda i: (i, 0)),
        out_shape=jax.ShapeDtypeStruct((N_NODES, D_PAD), f32),
    )(x.astype(f32), W1p)
    g, dis2 = pl.pallas_call(
        _scale_body,
        grid=grid,
        in_specs=[
            pl.BlockSpec((BR, NC), lambda i: (i, 0)),
            pl.BlockSpec((BR, D_PAD), lambda i: (i, 0)),
        ],
        out_specs=[
            pl.BlockSpec((BR, D_PAD), lambda i: (i, 0)),
            pl.BlockSpec((BR, 1), lambda i: (i, 0)),
        ],
        out_shape=[
            jax.ShapeDtypeStruct((N_NODES, D_PAD), f32),
            jax.ShapeDtypeStruct((N_NODES, 1), f32),
        ],
    )(deg_t, h)

    agg_parts, s_parts = _edge_kernel(src, dst, g, dis2.reshape(N_NODES))
    s_t = s_parts[:, :N_NODES].T

    out = pl.pallas_call(
        _fin_body,
        grid=grid,
        in_specs=[
            pl.BlockSpec((NC, BR, D_PAD), lambda i: (0, i, 0)),
            pl.BlockSpec((BR, D_PAD), lambda i: (i, 0)),
            pl.BlockSpec((BR, 1), lambda i: (i, 0)),
            pl.BlockSpec((BR, NC), lambda i: (i, 0)),
            pl.BlockSpec((1, D_PAD), lambda i: (0, 0)),
            pl.BlockSpec((D_PAD, N_CLASSES), lambda i: (0, 0)),
            pl.BlockSpec((1, N_CLASSES), lambda i: (0, 0)),
        ],
        out_specs=pl.BlockSpec((1, N_CLASSES), lambda i: (0, 0)),
        out_shape=jax.ShapeDtypeStruct((1, N_CLASSES), f32),
        scratch_shapes=[pltpu.VMEM((1, D_PAD), f32)],
    )(agg_parts, g, dis2, s_t, b1p, W2p, b2r)
    return out


# speedup vs baseline: 51.8949x; 1.0208x over previous
import functools

import jax
import jax.numpy as jnp
from jax import lax
from jax.experimental import pallas as pl
from jax.experimental.pallas import tpu as pltpu
from jax.experimental.pallas import tpu_sc as plsc

N_NODES = 10000
N_EDGES = 320000
D_IN = 128
D_HID = 100
D_PAD = 112
N_CLASSES = 10

NC = 2
NS = 16
NW = NC * NS
E_PER_W = N_EDGES // NW
CHUNK = 80
NCHUNK = E_PER_W // CHUNK
NPAD1 = 10240
STRIPE1 = NPAD1 // NS
RSTRIPE = N_NODES // NS
NBUF = 3
LOOKAHEAD = 2

_mesh = plsc.VectorSubcoreMesh(core_axis_name="c", subcore_axis_name="s")
_sc_params = pltpu.CompilerParams(needs_layout_passes=False,
                                  use_tc_tiling_on_sc=False)


@functools.partial(
    pl.kernel,
    out_type=jax.ShapeDtypeStruct((NC, NPAD1), jnp.float32),
    mesh=_mesh,
    compiler_params=_sc_params,
    scratch_types=[
        pltpu.VMEM((NCHUNK, CHUNK), jnp.int32),
        pltpu.VMEM((STRIPE1,), jnp.float32),
        pltpu.VMEM_SHARED((NPAD1,), jnp.float32),
    ],
)
def _deg_kernel(dst_hbm, out_hbm, dst_v, buf_v, deg_sh):
    cid = lax.axis_index("c")
    sid = lax.axis_index("s")
    wid = cid * NS + sid

    def zero_body(j, _):
        buf_v[pl.ds(j * 16, 16)] = jnp.zeros((16,), jnp.float32)
        return 0

    lax.fori_loop(0, STRIPE1 // 16, zero_body, 0)
    pltpu.sync_copy(buf_v, deg_sh.at[pl.ds(sid * STRIPE1, STRIPE1)])
    pltpu.sync_copy(dst_hbm.at[wid], dst_v)

    def ones_body(j, _):
        buf_v[pl.ds(j * 16, 16)] = jnp.full((16,), 1.0, jnp.float32)
        return 0

    lax.fori_loop(0, CHUNK // 16, ones_body, 0)
    plsc.subcore_barrier()

    def edge_body(ci, _):
        pltpu.sync_copy(buf_v.at[pl.ds(0, CHUNK)], deg_sh.at[dst_v.at[ci]],
                        add=True)
        return 0

    lax.fori_loop(0, NCHUNK, edge_body, 0)
    plsc.subcore_barrier()
    pltpu.sync_copy(deg_sh.at[pl.ds(sid * STRIPE1, STRIPE1)],
                    out_hbm.at[cid, pl.ds(sid * STRIPE1, STRIPE1)])


def _mm_body(x_ref, w1_ref, h_ref):
    h_ref[...] = jnp.dot(x_ref[...], w1_ref[...],
                         preferred_element_type=jnp.float32)


def _scale_body(degt_ref, h_ref, g_ref, dis_ref):
    deg = degt_ref[:, 0] + degt_ref[:, 1] + 1.0
    dis = lax.rsqrt(deg)
    g_ref[...] = h_ref[...] * dis[:, None]
    dis_ref[...] = dis[:, None]


@functools.partial(
    pl.kernel,
    out_type=[
        jax.ShapeDtypeStruct((NC, N_NODES, D_PAD), jnp.float32),
        jax.ShapeDtypeStruct((NC, NPAD1), jnp.float32),
    ],
    mesh=_mesh,
    compiler_params=_sc_params,
    scratch_types=[
        pltpu.VMEM((NCHUNK, CHUNK), jnp.int32),
        pltpu.VMEM((NCHUNK, CHUNK), jnp.int32),
        pltpu.VMEM((N_NODES,), jnp.float32),
        [pltpu.VMEM((CHUNK, D_PAD), jnp.float32) for _ in range(NBUF)],
        [pltpu.VMEM((CHUNK,), jnp.float32) for _ in range(NBUF)],
        pltpu.VMEM((STRIPE1,), jnp.float32),
        pltpu.VMEM_SHARED((N_NODES, D_PAD), jnp.float32),
        pltpu.VMEM_SHARED((NPAD1,), jnp.float32),
        [pltpu.SemaphoreType.DMA for _ in range(NBUF)],
        [pltpu.SemaphoreType.DMA for _ in range(NBUF)],
    ],
)
def _edge_kernel(src_hbm, dst_hbm, g_hbm, dis_hbm, agg_out, s_out,
                 src_v, dst_v, dis_v, bufs, vals, zbuf_v,
                 agg_sh, s_sh, gsem, ssem):
    cid = lax.axis_index("c")
    sid = lax.axis_index("s")
    wid = cid * NS + sid

    def zrow_body(r, _):
        for k in range(D_PAD // 16):
            bufs[0][r, pl.ds(k * 16, 16)] = jnp.zeros((16,), jnp.float32)
        return 0

    lax.fori_loop(0, CHUNK, zrow_body, 0)

    def zs_body(j, _):
        zbuf_v[pl.ds(j * 16, 16)] = jnp.zeros((16,), jnp.float32)
        return 0

    lax.fori_loop(0, STRIPE1 // 16, zs_body, 0)

    for b in range(RSTRIPE // CHUNK):
        pltpu.sync_copy(bufs[0],
                        agg_sh.at[pl.ds(sid * RSTRIPE + b * CHUNK, CHUNK)])
    _tail = RSTRIPE - (RSTRIPE // CHUNK) * CHUNK
    pltpu.sync_copy(
        bufs[0].at[pl.ds(0, _tail)],
        agg_sh.at[pl.ds(sid * RSTRIPE + (RSTRIPE // CHUNK) * CHUNK, _tail)])
    pltpu.sync_copy(zbuf_v, s_sh.at[pl.ds(sid * STRIPE1, STRIPE1)])

    pltpu.sync_copy(src_hbm.at[wid], src_v)
    pltpu.sync_copy(dst_hbm.at[wid], dst_v)
    pltpu.sync_copy(dis_hbm, dis_v)
    plsc.subcore_barrier()

    def _wait_pair(b):
        pltpu.make_async_copy(bufs[b], agg_sh.at[dst_v.at[0]], ssem[b]).wait()
        pltpu.make_async_copy(vals[b], s_sh.at[src_v.at[0]], ssem[b]).wait()

    def _consume(c, b):
        pltpu.make_async_copy(g_hbm.at[src_v.at[c]], bufs[b], gsem[b]).wait()
        for j in range(CHUNK // 16):
            d16 = dst_v[c, pl.ds(j * 16, 16)]
            vals[b][pl.ds(j * 16, 16)] = plsc.load_gather(dis_v, [d16])
        pltpu.async_copy(bufs[b], agg_sh.at[dst_v.at[c]], ssem[b], add=True)
        pltpu.async_copy(vals[b], s_sh.at[src_v.at[c]], ssem[b], add=True)

    for c in range(LOOKAHEAD):
        pltpu.async_copy(g_hbm.at[src_v.at[c]], bufs[c % NBUF], gsem[c % NBUF])

    def edge_body(k, _):
        for b in range(NBUF):
            c = k * NBUF + b
            _consume(c, b)
            cn = c + LOOKAHEAD
            bn = (b + LOOKAHEAD) % NBUF

            @pl.when(cn < NCHUNK)
            def _():
                @pl.when(cn >= NBUF)
                def _():
                    _wait_pair(bn)

                pltpu.async_copy(g_hbm.at[src_v.at[cn]], bufs[bn], gsem[bn])

        return 0

    nfull = NCHUNK // NBUF
    lax.fori_loop(0, nfull, edge_body, 0)
    for b in range(NCHUNK - nfull * NBUF):
        _consume(nfull * NBUF + b, b)
    for b in range(NBUF):
        _wait_pair(b)
    plsc.subcore_barrier()

    sl = pl.ds(sid * RSTRIPE, RSTRIPE)
    pltpu.sync_copy(agg_sh.at[sl], agg_out.at[cid, sl])
    pltpu.sync_copy(s_sh.at[pl.ds(sid * STRIPE1, STRIPE1)],
                    s_out.at[cid, pl.ds(sid * STRIPE1, STRIPE1)])


def _fin_body(aggp_ref, g_ref, dis_ref, st_ref, b1_ref, w2_ref, b2_ref,
              out_ref, acc_ref):
    i = pl.program_id(0)
    nb = pl.num_programs(0)
    agg = aggp_ref[0] + aggp_ref[1] + g_ref[...]
    dis = dis_ref[...]
    h1 = jnp.maximum(agg * dis + b1_ref[...], 0.0)
    srow = st_ref[:, 0] + st_ref[:, 1]
    c = dis[:, 0] * (srow + dis[:, 0])
    contrib = jnp.dot(c[None, :], h1, preferred_element_type=jnp.float32)

    @pl.when(i == 0)
    def _():
        acc_ref[...] = contrib

    @pl.when(i > 0)
    def _():
        acc_ref[...] = acc_ref[...] + contrib

    @pl.when(i == nb - 1)
    def _():
        v = acc_ref[...]
        pooled = (jnp.dot(v, w2_ref[...], preferred_element_type=jnp.float32)
                  / float(N_NODES)) + b2_ref[...]
        m = jnp.max(pooled, axis=1, keepdims=True)
        lse = jnp.log(jnp.sum(jnp.exp(pooled - m), axis=1, keepdims=True)) + m
        out_ref[...] = pooled - lse


def kernel(x, edge_index, W1, b1, W2, b2):
    f32 = jnp.float32
    src = edge_index[0].astype(jnp.int32).reshape(NW, NCHUNK, CHUNK)
    dst = edge_index[1].astype(jnp.int32).reshape(NW, NCHUNK, CHUNK)
    W1p = jnp.pad(W1.astype(f32), ((0, 0), (0, D_PAD - D_HID)))
    b1p = jnp.pad(b1.astype(f32), (0, D_PAD - D_HID)).reshape(1, D_PAD)
    W2p = jnp.pad(W2.astype(f32), ((0, D_PAD - D_HID), (0, 0)))
    b2r = b2.astype(f32).reshape(1, N_CLASSES)

    deg_parts = _deg_kernel(dst)
    deg_t = deg_parts[:, :N_NODES].T

    BR = 2000
    grid = (N_NODES // BR,)
    h = pl.pallas_call(
        _mm_body,
        grid=grid,
        in_specs=[
            pl.BlockSpec((BR, D_IN), lambda i: (i, 0)),
            pl.BlockSpec((D_IN, D_PAD), lambda i: (0, 0)),
        ],
        out_specs=pl.BlockSpec((BR, D_PAD), lambda i: (i, 0)),
        out_shape=jax.ShapeDtypeStruct((N_NODES, D_PAD), f32),
    )(x.astype(f32), W1p)
    g, dis2 = pl.pallas_call(
        _scale_body,
        grid=grid,
        in_specs=[
            pl.BlockSpec((BR, NC), lambda i: (i, 0)),
            pl.BlockSpec((BR, D_PAD), lambda i: (i, 0)),
        ],
        out_specs=[
            pl.BlockSpec((BR, D_PAD), lambda i: (i, 0)),
            pl.BlockSpec((BR, 1), lambda i: (i, 0)),
        ],
        out_shape=[
            jax.ShapeDtypeStruct((N_NODES, D_PAD), f32),
            jax.ShapeDtypeStruct((N_NODES, 1), f32),
        ],
    )(deg_t, h)

    agg_parts, s_parts = _edge_kernel(src, dst, g, dis2.reshape(N_NODES))
    s_t = s_parts[:, :N_NODES].T

    out = pl.pallas_call(
        _fin_body,
        grid=grid,
        in_specs=[
            pl.BlockSpec((NC, BR, D_PAD), lambda i: (0, i, 0)),
            pl.BlockSpec((BR, D_PAD), lambda i: (i, 0)),
            pl.BlockSpec((BR, 1), lambda i: (i, 0)),
            pl.BlockSpec((BR, NC), lambda i: (i, 0)),
            pl.BlockSpec((1, D_PAD), lambda i: (0, 0)),
            pl.BlockSpec((D_PAD, N_CLASSES), lambda i: (0, 0)),
            pl.BlockSpec((1, N_CLASSES), lambda i: (0, 0)),
        ],
        out_specs=pl.BlockSpec((1, N_CLASSES), lambda i: (0, 0)),
        out_shape=jax.ShapeDtypeStruct((1, N_CLASSES), f32),
        scratch_shapes=[pltpu.VMEM((1, D_PAD), f32)],
    )(agg_parts, g, dis2, s_t, b1p, W2p, b2r)
    return out
